# baseline (speedup 1.0000x reference)
import functools

import jax
import jax.numpy as jnp
from jax import lax
from jax.experimental import pallas as pl
from jax.experimental.pallas import tpu as pltpu
from jax.experimental.pallas import tpu_sc as plsc

E = 3_200_000
D = 16
N = 100_000
C = 1024
NCHUNK = E // C
NW = 32
BLK = 1000
NBLK = N // BLK

_mesh = plsc.VectorSubcoreMesh(core_axis_name="c", subcore_axis_name="s")
_sc_params = pltpu.CompilerParams(
    use_tc_tiling_on_sc=False, needs_layout_passes=False)


@functools.partial(
    pl.kernel,
    out_type=jax.ShapeDtypeStruct((2, N, D), jnp.float32),
    mesh=_mesh,
    compiler_params=_sc_params,
    scratch_types=[
        pltpu.MemorySpace.VMEM_SHARED((N, D), jnp.float32),
        pltpu.VMEM((C, D), jnp.float32),
        pltpu.VMEM((C, 1), jnp.float32),
        pltpu.VMEM((C,), jnp.int32),
        pltpu.VMEM((16,), jnp.float32),
    ],
)
def _scatter_kernel(sh_hbm, cut_hbm, recv_hbm, inv_hbm, out_hbm,
                    acc, sh_buf, cut_buf, idx_buf, inv_buf):
    c = lax.axis_index("c")
    s = lax.axis_index("s")
    wid = c * 16 + s

    pltpu.sync_copy(inv_hbm, inv_buf)
    inv_vec = inv_buf[:]

    stage = sh_buf.at[pl.ds(0, BLK)]

    def _zero(i, _):
        sh_buf[i, :] = jnp.zeros((D,), jnp.float32)
        return _

    lax.fori_loop(0, BLK, _zero, 0)
    nzb = (NBLK - s + 15) // 16

    def _zero_blk(i, _):
        r0 = pl.multiple_of((s + i * 16) * BLK, 8)
        pltpu.sync_copy(stage, acc.at[pl.ds(r0, BLK)])
        return _

    lax.fori_loop(0, nzb, _zero_blk, 0)
    plsc.subcore_barrier()

    nmine = (NCHUNK - wid + NW - 1) // NW

    def chunk_body(i, _):
        chunk = wid + i * NW
        base = pl.multiple_of(chunk * C, 8)
        pltpu.sync_copy(sh_hbm.at[pl.ds(base, C)], sh_buf)
        pltpu.sync_copy(cut_hbm.at[pl.ds(base, C)], cut_buf)
        pltpu.sync_copy(recv_hbm.at[pl.ds(base, C)], idx_buf)

        zeros16 = jnp.zeros((16,), jnp.int32)
        iota16 = lax.iota(jnp.int32, 16)

        def scale_grp(g, _):
            cutv = plsc.load_gather(cut_buf, [g * 16 + iota16, zeros16])
            cutv = cutv * inv_vec
            base_r = g * 16
            for k in range(16):
                sh_buf[base_r + k, :] = sh_buf[base_r + k, :] * cutv[k]
            return _

        lax.fori_loop(0, C // 16, scale_grp, 0)

        pltpu.sync_copy(sh_buf, acc.at[idx_buf], add=True)
        return _

    lax.fori_loop(0, nmine, chunk_body, 0)
    plsc.subcore_barrier()

    def _write_blk(i, _):
        r0 = pl.multiple_of((s + i * 16) * BLK, 8)
        pltpu.sync_copy(acc.at[pl.ds(r0, BLK)], stage)
        pltpu.sync_copy(stage, out_hbm.at[c, pl.ds(r0, BLK)])
        return _

    lax.fori_loop(0, nzb, _write_blk, 0)


@functools.partial(
    pl.kernel,
    out_type=jax.ShapeDtypeStruct((N, D), jnp.float32),
    mesh=_mesh,
    compiler_params=_sc_params,
    scratch_types=[
        pltpu.VMEM((BLK, D), jnp.float32),
        pltpu.VMEM((BLK, D), jnp.float32),
        pltpu.VMEM((16,), jnp.float32),
    ],
)
def _combine_kernel(part_hbm, off_hbm, out_hbm, b0, b1, off_buf):
    c = lax.axis_index("c")
    s = lax.axis_index("s")
    wid = c * 16 + s
    pltpu.sync_copy(off_hbm, off_buf)
    off_vec = off_buf[:]
    nb = (NBLK - wid + NW - 1) // NW

    def blk_body(i, _):
        r0 = pl.multiple_of((wid + i * NW) * BLK, 8)
        pltpu.sync_copy(part_hbm.at[0, pl.ds(r0, BLK)], b0)
        pltpu.sync_copy(part_hbm.at[1, pl.ds(r0, BLK)], b1)

        def add_row(r, _):
            b0[r, :] = b0[r, :] + b1[r, :] + off_vec
            return _

        lax.fori_loop(0, BLK, add_row, 0)
        pltpu.sync_copy(b0, out_hbm.at[pl.ds(r0, BLK)])
        return _

    lax.fori_loop(0, nb, blk_body, 0)


def kernel(sh_vectors, cutoffs, receivers, avg_num_neighbors, num_nodes):
    cut = cutoffs
    recv = receivers.astype(jnp.int32)
    inv = jnp.full((16,), 1.0, jnp.float32) / jnp.asarray(
        avg_num_neighbors, jnp.float32)
    off = jnp.full((16,), num_nodes - N, jnp.float32)
    partials = _scatter_kernel(sh_vectors, cut, recv, inv)
    return _combine_kernel(partials, off)


# speedup vs baseline: 6.0538x; 6.0538x over previous
import functools

import jax
import jax.numpy as jnp
from jax import lax
from jax.experimental import pallas as pl
from jax.experimental.pallas import tpu as pltpu
from jax.experimental.pallas import tpu_sc as plsc

E = 3_200_000
D = 16
N = 100_000
C = 640
NCHUNK = E // C
NW = 32
BLK = 400
NBLK = N // BLK

_mesh = plsc.VectorSubcoreMesh(core_axis_name="c", subcore_axis_name="s")
_sc_params = pltpu.CompilerParams(
    use_tc_tiling_on_sc=False, needs_layout_passes=False)


@functools.partial(
    pl.kernel,
    out_type=jax.ShapeDtypeStruct((2, N, D), jnp.float32),
    mesh=_mesh,
    compiler_params=_sc_params,
    scratch_types=[
        pltpu.MemorySpace.VMEM_SHARED((N, D), jnp.float32),
        pltpu.VMEM((C, D), jnp.float32),
        pltpu.VMEM((2, C // 128, 8, 128), jnp.float32),
        pltpu.VMEM((C,), jnp.float32),
        pltpu.VMEM((C // 128, 128), jnp.int32),
        pltpu.VMEM((16,), jnp.float32),
    ],
)
def _scatter_kernel(sh_hbm, cut_hbm, recv_hbm, inv_hbm, out_hbm,
                    acc, sh_buf, shT_buf, cut_buf, idx_buf, inv_buf):
    c = lax.axis_index("c")
    s = lax.axis_index("s")
    wid = c * 16 + s

    pltpu.sync_copy(inv_hbm, inv_buf)
    inv_vec = inv_buf[:]

    stage = sh_buf.at[pl.ds(0, BLK)]

    def _zero(i, _):
        sh_buf[i, :] = jnp.zeros((D,), jnp.float32)
        return _

    lax.fori_loop(0, BLK, _zero, 0)
    nzb = (NBLK - s + 15) // 16

    def _zero_blk(i, _):
        r0 = pl.multiple_of((s + i * 16) * BLK, 8)
        pltpu.sync_copy(stage, acc.at[pl.ds(r0, BLK)])
        return _

    lax.fori_loop(0, nzb, _zero_blk, 0)
    plsc.subcore_barrier()

    nmine = (NCHUNK - wid + NW - 1) // NW

    def chunk_body(i, _):
        chunk = wid + i * NW
        base = pl.multiple_of(chunk * C, 8)
        et0 = chunk * (C // 128)
        pltpu.sync_copy(sh_hbm.at[0, pl.ds(et0, C // 128)], shT_buf.at[0])
        pltpu.sync_copy(sh_hbm.at[1, pl.ds(et0, C // 128)], shT_buf.at[1])
        pltpu.sync_copy(cut_hbm.at[pl.ds(base, C)], cut_buf)
        pltpu.sync_copy(recv_hbm.at[chunk], idx_buf)

        iota16 = lax.iota(jnp.int32, 16)

        def scale_grp(g, _):
            ct = g // 8
            doff = (g % 8) * 16
            cutv = cut_buf[pl.ds(g * 16, 16)] * inv_vec
            e_vec = g * 16 + iota16
            for f in range(16):
                v = shT_buf[f // 8, ct, f % 8, pl.ds(doff, 16)] * cutv
                plsc.store_scatter(
                    sh_buf, [e_vec, jnp.full((16,), f, jnp.int32)], v)
            return _

        lax.fori_loop(0, C // 16, scale_grp, 0)

        for j in range(C // 128):
            pltpu.sync_copy(
                sh_buf.at[pl.ds(j * 128, 128)],
                acc.at[idx_buf.at[j]],
                add=True,
            )
        return _

    lax.fori_loop(0, nmine, chunk_body, 0)
    plsc.subcore_barrier()

    def _write_blk(i, _):
        r0 = pl.multiple_of((s + i * 16) * BLK, 8)
        pltpu.sync_copy(acc.at[pl.ds(r0, BLK)], stage)
        pltpu.sync_copy(stage, out_hbm.at[c, pl.ds(r0, BLK)])
        return _

    lax.fori_loop(0, nzb, _write_blk, 0)


@functools.partial(
    pl.kernel,
    out_type=jax.ShapeDtypeStruct((N, D), jnp.float32),
    mesh=_mesh,
    compiler_params=_sc_params,
    scratch_types=[
        pltpu.VMEM((BLK, D), jnp.float32),
        pltpu.VMEM((BLK, D), jnp.float32),
        pltpu.VMEM((16,), jnp.float32),
    ],
)
def _combine_kernel(part_hbm, off_hbm, out_hbm, b0, b1, off_buf):
    c = lax.axis_index("c")
    s = lax.axis_index("s")
    wid = c * 16 + s
    pltpu.sync_copy(off_hbm, off_buf)
    off_vec = off_buf[:]
    nb = (NBLK - wid + NW - 1) // NW

    def blk_body(i, _):
        r0 = pl.multiple_of((wid + i * NW) * BLK, 8)
        pltpu.sync_copy(part_hbm.at[0, pl.ds(r0, BLK)], b0)
        pltpu.sync_copy(part_hbm.at[1, pl.ds(r0, BLK)], b1)

        def add_row(r, _):
            b0[r, :] = b0[r, :] + b1[r, :] + off_vec
            return _

        lax.fori_loop(0, BLK, add_row, 0)
        pltpu.sync_copy(b0, out_hbm.at[pl.ds(r0, BLK)])
        return _

    lax.fori_loop(0, nb, blk_body, 0)


def kernel(sh_vectors, cutoffs, receivers, avg_num_neighbors, num_nodes):
    sh4 = jnp.transpose(
        jnp.reshape(jnp.transpose(sh_vectors), (2, 8, E // 128, 128)),
        (0, 2, 1, 3))
    cut = jnp.reshape(cutoffs, (E,))
    recv = jnp.reshape(receivers.astype(jnp.int32), (NCHUNK, C // 128, 128))
    inv = jnp.full((16,), 1.0, jnp.float32) / jnp.asarray(
        avg_num_neighbors, jnp.float32)
    off = jnp.full((16,), num_nodes - N, jnp.float32)
    partials = _scatter_kernel(sh4, cut, recv, inv)
    return _combine_kernel(partials, off)


# speedup vs baseline: 10.2529x; 1.6936x over previous
import functools

import jax
import jax.numpy as jnp
from jax import lax
from jax.experimental import pallas as pl
from jax.experimental.pallas import tpu as pltpu
from jax.experimental.pallas import tpu_sc as plsc

E = 3_200_000
D = 16
N = 100_000
C = 512
NCHUNK = E // C
CT = C // 128
NW = 32
BLK = 400
NBLK = N // BLK

_mesh = plsc.VectorSubcoreMesh(core_axis_name="c", subcore_axis_name="s")
_sc_params = pltpu.CompilerParams(
    use_tc_tiling_on_sc=False, needs_layout_passes=False)


@functools.partial(
    pl.kernel,
    out_type=jax.ShapeDtypeStruct((2, N, D), jnp.float32),
    mesh=_mesh,
    compiler_params=_sc_params,
    scratch_types=[
        pltpu.MemorySpace.VMEM_SHARED((N, D), jnp.float32),
        pltpu.VMEM((C, D), jnp.float32),
        pltpu.VMEM((2, 2, CT, 8, 128), jnp.float32),
        pltpu.VMEM((2, C), jnp.float32),
        pltpu.VMEM((2, CT, 128), jnp.int32),
        pltpu.VMEM((16,), jnp.float32),
        pltpu.SemaphoreType.DMA,
        pltpu.SemaphoreType.DMA,
        pltpu.SemaphoreType.DMA,
    ],
)
def _scatter_kernel(sh_hbm, cut_hbm, recv_hbm, inv_hbm, out_hbm,
                    acc, sh_buf, shT_buf, cut_buf, idx_buf, inv_buf,
                    in_sem0, in_sem1, scat_sem):
    c = lax.axis_index("c")
    s = lax.axis_index("s")
    wid = c * 16 + s

    pltpu.sync_copy(inv_hbm, inv_buf)
    inv_vec = inv_buf[:]

    stage = sh_buf.at[pl.ds(0, BLK)]

    def _zero(i, _):
        sh_buf[i, :] = jnp.zeros((D,), jnp.float32)
        return _

    lax.fori_loop(0, BLK, _zero, 0)
    nzb = (NBLK - s + 15) // 16

    def _zero_blk(i, _):
        r0 = pl.multiple_of((s + i * 16) * BLK, 8)
        pltpu.sync_copy(stage, acc.at[pl.ds(r0, BLK)])
        return _

    lax.fori_loop(0, nzb, _zero_blk, 0)
    plsc.subcore_barrier()

    nmine = (NCHUNK - wid + NW - 1) // NW
    in_sems = (in_sem0, in_sem1)
    iota16 = lax.iota(jnp.int32, 16)

    def issue_in(i, b):
        chunk = wid + i * NW
        base = pl.multiple_of(chunk * C, 8)
        pltpu.async_copy(sh_hbm.at[:, pl.ds(chunk * CT, CT)],
                         shT_buf.at[b], in_sems[b])
        pltpu.async_copy(cut_hbm.at[pl.ds(base, C)],
                         cut_buf.at[b], in_sems[b])
        pltpu.async_copy(recv_hbm.at[chunk], idx_buf.at[b], in_sems[b])

    def wait_in(i, b):
        chunk = wid + i * NW
        base = pl.multiple_of(chunk * C, 8)
        pltpu.make_async_copy(sh_hbm.at[:, pl.ds(chunk * CT, CT)],
                              shT_buf.at[b], in_sems[b]).wait()
        pltpu.make_async_copy(cut_hbm.at[pl.ds(base, C)],
                              cut_buf.at[b], in_sems[b]).wait()
        pltpu.make_async_copy(recv_hbm.at[chunk], idx_buf.at[b],
                              in_sems[b]).wait()

    def wait_scat(b):
        for j in range(CT):
            pltpu.make_async_copy(
                sh_buf.at[pl.ds(j * 128, 128)],
                acc.at[idx_buf.at[b, j]], scat_sem).wait()

    def process(i, b):
        def scale_grp(g, _):
            ct = g // 8
            doff = (g % 8) * 16
            cutv = cut_buf[b, pl.ds(g * 16, 16)] * inv_vec
            e_vec = g * 16 + iota16
            for f in range(16):
                v = shT_buf[b, f // 8, ct, f % 8, pl.ds(doff, 16)] * cutv
                plsc.store_scatter(
                    sh_buf, [e_vec, jnp.full((16,), f, jnp.int32)], v)
            return _

        lax.fori_loop(0, C // 16, scale_grp, 0)

        for j in range(CT):
            pltpu.async_copy(
                sh_buf.at[pl.ds(j * 128, 128)],
                acc.at[idx_buf.at[b, j]], scat_sem, add=True)

    issue_in(0, 0)

    def pair_body(k, carry):
        for b in range(2):
            i = 2 * k + b

            @pl.when(i < nmine)
            def _():
                wait_in(i, b)

                @pl.when(i > 0)
                def _():
                    wait_scat(1 - b)

                @pl.when(i + 1 < nmine)
                def _():
                    issue_in(i + 1, 1 - b)

                process(i, b)
        return carry

    lax.fori_loop(0, (nmine + 1) // 2, pair_body, 0)
    last_b = (nmine - 1) % 2
    wait_scat(last_b)
    plsc.subcore_barrier()

    def _write_blk(i, _):
        r0 = pl.multiple_of((s + i * 16) * BLK, 8)
        pltpu.sync_copy(acc.at[pl.ds(r0, BLK)], stage)
        pltpu.sync_copy(stage, out_hbm.at[c, pl.ds(r0, BLK)])
        return _

    lax.fori_loop(0, nzb, _write_blk, 0)


@functools.partial(
    pl.kernel,
    out_type=jax.ShapeDtypeStruct((N, D), jnp.float32),
    mesh=_mesh,
    compiler_params=_sc_params,
    scratch_types=[
        pltpu.VMEM((BLK, D), jnp.float32),
        pltpu.VMEM((BLK, D), jnp.float32),
        pltpu.VMEM((16,), jnp.float32),
    ],
)
def _combine_kernel(part_hbm, off_hbm, out_hbm, b0, b1, off_buf):
    c = lax.axis_index("c")
    s = lax.axis_index("s")
    wid = c * 16 + s
    pltpu.sync_copy(off_hbm, off_buf)
    off_vec = off_buf[:]
    nb = (NBLK - wid + NW - 1) // NW

    def blk_body(i, _):
        r0 = pl.multiple_of((wid + i * NW) * BLK, 8)
        pltpu.sync_copy(part_hbm.at[0, pl.ds(r0, BLK)], b0)
        pltpu.sync_copy(part_hbm.at[1, pl.ds(r0, BLK)], b1)

        def add_row(r, _):
            b0[r, :] = b0[r, :] + b1[r, :] + off_vec
            return _

        lax.fori_loop(0, BLK, add_row, 0)
        pltpu.sync_copy(b0, out_hbm.at[pl.ds(r0, BLK)])
        return _

    lax.fori_loop(0, nb, blk_body, 0)


def kernel(sh_vectors, cutoffs, receivers, avg_num_neighbors, num_nodes):
    sh4 = jnp.transpose(
        jnp.reshape(jnp.transpose(sh_vectors), (2, 8, E // 128, 128)),
        (0, 2, 1, 3))
    cut = jnp.reshape(cutoffs, (E,))
    recv = jnp.reshape(receivers.astype(jnp.int32), (NCHUNK, C // 128, 128))
    inv = jnp.full((16,), 1.0, jnp.float32) / jnp.asarray(
        avg_num_neighbors, jnp.float32)
    off = jnp.full((16,), num_nodes - N, jnp.float32)
    partials = _scatter_kernel(sh4, cut, recv, inv)
    return _combine_kernel(partials, off)


# speedup vs baseline: 21.1039x; 2.0583x over previous
import functools

import jax
import jax.numpy as jnp
from jax import lax
from jax.experimental import pallas as pl
from jax.experimental.pallas import tpu as pltpu
from jax.experimental.pallas import tpu_sc as plsc

E = 3_200_000
D = 16
N = 100_000
C = 512
NCHUNK = E // C
CT = C // 128
NW = 32
BLK = 400
NBLK = N // BLK

_mesh = plsc.VectorSubcoreMesh(core_axis_name="c", subcore_axis_name="s")
_sc_params = pltpu.CompilerParams(
    use_tc_tiling_on_sc=False, needs_layout_passes=False)


@functools.partial(
    pl.kernel,
    out_type=jax.ShapeDtypeStruct((2, N, D), jnp.float32),
    mesh=_mesh,
    compiler_params=_sc_params,
    scratch_types=[
        pltpu.MemorySpace.VMEM_SHARED((N, D), jnp.float32),
        pltpu.VMEM((C, D), jnp.float32),
        pltpu.VMEM((2, 2, CT, 8, 128), jnp.float32),
        pltpu.VMEM((2, C), jnp.float32),
        pltpu.VMEM((2, CT, 128), jnp.int32),
        pltpu.VMEM((16,), jnp.float32),
        pltpu.SemaphoreType.DMA,
        pltpu.SemaphoreType.DMA,
        pltpu.SemaphoreType.DMA,
    ],
)
def _scatter_kernel(sh_hbm, cut_hbm, recv_hbm, inv_hbm, out_hbm,
                    acc, sh_buf, shT_buf, cut_buf, idx_buf, inv_buf,
                    in_sem0, in_sem1, scat_sem):
    c = lax.axis_index("c")
    s = lax.axis_index("s")
    wid = c * 16 + s

    pltpu.sync_copy(inv_hbm, inv_buf)
    inv_vec = inv_buf[:]

    stage = sh_buf.at[pl.ds(0, BLK)]

    def _zero(i, _):
        sh_buf[i, :] = jnp.zeros((D,), jnp.float32)
        return _

    lax.fori_loop(0, BLK, _zero, 0)
    nzb = (NBLK - s + 15) // 16

    def _zero_blk(i, _):
        r0 = pl.multiple_of((s + i * 16) * BLK, 8)
        pltpu.sync_copy(stage, acc.at[pl.ds(r0, BLK)])
        return _

    lax.fori_loop(0, nzb, _zero_blk, 0)
    plsc.subcore_barrier()

    nmine = (NCHUNK - wid + NW - 1) // NW
    in_sems = (in_sem0, in_sem1)
    iota16 = lax.iota(jnp.int32, 16)

    def issue_in(i, b):
        chunk = wid + i * NW
        base = pl.multiple_of(chunk * C, 8)
        pltpu.async_copy(sh_hbm.at[:, pl.ds(chunk * CT, CT)],
                         shT_buf.at[b], in_sems[b])
        pltpu.async_copy(cut_hbm.at[pl.ds(base, C)],
                         cut_buf.at[b], in_sems[b])
        pltpu.async_copy(recv_hbm.at[chunk], idx_buf.at[b], in_sems[b])

    def wait_in(i, b):
        chunk = wid + i * NW
        base = pl.multiple_of(chunk * C, 8)
        pltpu.make_async_copy(sh_hbm.at[:, pl.ds(chunk * CT, CT)],
                              shT_buf.at[b], in_sems[b]).wait()
        pltpu.make_async_copy(cut_hbm.at[pl.ds(base, C)],
                              cut_buf.at[b], in_sems[b]).wait()
        pltpu.make_async_copy(recv_hbm.at[chunk], idx_buf.at[b],
                              in_sems[b]).wait()

    def wait_scat(b):
        for j in range(CT):
            pltpu.make_async_copy(
                sh_buf.at[pl.ds(j * 128, 128)],
                acc.at[idx_buf.at[b, j]], scat_sem).wait()

    def process(i, b):
        @functools.partial(plsc.parallel_loop, 0, C // 16, unroll=4)
        def scale_grp(g):
            ct = g // 8
            doff = (g % 8) * 16
            cutv = cut_buf[b, pl.ds(g * 16, 16)] * inv_vec
            e_vec = g * 16 + iota16
            for f in range(16):
                v = shT_buf[b, f // 8, ct, f % 8, pl.ds(doff, 16)] * cutv
                plsc.store_scatter(
                    sh_buf, [e_vec, jnp.full((16,), f, jnp.int32)], v)

        for j in range(CT):
            pltpu.async_copy(
                sh_buf.at[pl.ds(j * 128, 128)],
                acc.at[idx_buf.at[b, j]], scat_sem, add=True)

    issue_in(0, 0)

    def pair_body(k, carry):
        for b in range(2):
            i = 2 * k + b

            @pl.when(i < nmine)
            def _():
                wait_in(i, b)

                @pl.when(i > 0)
                def _():
                    wait_scat(1 - b)

                @pl.when(i + 1 < nmine)
                def _():
                    issue_in(i + 1, 1 - b)

                process(i, b)
        return carry

    lax.fori_loop(0, (nmine + 1) // 2, pair_body, 0)
    last_b = (nmine - 1) % 2
    wait_scat(last_b)
    plsc.subcore_barrier()

    def _write_blk(i, _):
        r0 = pl.multiple_of((s + i * 16) * BLK, 8)
        pltpu.sync_copy(acc.at[pl.ds(r0, BLK)], stage)
        pltpu.sync_copy(stage, out_hbm.at[c, pl.ds(r0, BLK)])
        return _

    lax.fori_loop(0, nzb, _write_blk, 0)


@functools.partial(
    pl.kernel,
    out_type=jax.ShapeDtypeStruct((N, D), jnp.float32),
    mesh=_mesh,
    compiler_params=_sc_params,
    scratch_types=[
        pltpu.VMEM((BLK, D), jnp.float32),
        pltpu.VMEM((BLK, D), jnp.float32),
        pltpu.VMEM((16,), jnp.float32),
    ],
)
def _combine_kernel(part_hbm, off_hbm, out_hbm, b0, b1, off_buf):
    c = lax.axis_index("c")
    s = lax.axis_index("s")
    wid = c * 16 + s
    pltpu.sync_copy(off_hbm, off_buf)
    off_vec = off_buf[:]
    nb = (NBLK - wid + NW - 1) // NW

    def blk_body(i, _):
        r0 = pl.multiple_of((wid + i * NW) * BLK, 8)
        pltpu.sync_copy(part_hbm.at[0, pl.ds(r0, BLK)], b0)
        pltpu.sync_copy(part_hbm.at[1, pl.ds(r0, BLK)], b1)

        def add_row(r, _):
            b0[r, :] = b0[r, :] + b1[r, :] + off_vec
            return _

        lax.fori_loop(0, BLK, add_row, 0)
        pltpu.sync_copy(b0, out_hbm.at[pl.ds(r0, BLK)])
        return _

    lax.fori_loop(0, nb, blk_body, 0)


def kernel(sh_vectors, cutoffs, receivers, avg_num_neighbors, num_nodes):
    sh4 = jnp.transpose(
        jnp.reshape(jnp.transpose(sh_vectors), (2, 8, E // 128, 128)),
        (0, 2, 1, 3))
    cut = jnp.reshape(cutoffs, (E,))
    recv = jnp.reshape(receivers.astype(jnp.int32), (NCHUNK, C // 128, 128))
    inv = jnp.full((16,), 1.0, jnp.float32) / jnp.asarray(
        avg_num_neighbors, jnp.float32)
    off = jnp.full((16,), num_nodes - N, jnp.float32)
    partials = _scatter_kernel(sh4, cut, recv, inv)
    return _combine_kernel(partials, off)
